# baseline (speedup 1.0000x reference)
import jax
import jax.numpy as jnp
from jax import lax
from jax.experimental import pallas as pl
from jax.experimental.pallas import tpu as pltpu
from jax.experimental.pallas import tpu_sc as plsc

N = 10000
E = 320000
DF = 128
DE = 16
H = 64
G = 64

NC = 2
NS = 16
NW = NC * NS
EPW = E // NW
CK = 128
NCH = 80
PAD = NCH * CK - EPW
ND = N + 16
FE = EPW // CK
TAIL = EPW - FE * CK
RPS = 624
BR = 1000
NB = N // BR
BE = 4000
NBE = E // BE

_mesh = plsc.VectorSubcoreMesh(core_axis_name="c", subcore_axis_name="s")
_f32 = jnp.float32
_sc_params = pltpu.CompilerParams(use_tc_tiling_on_sc=False)
_PH = lax.Precision.HIGHEST


def _bf16r(v):
    return v.astype(jnp.bfloat16).astype(_f32)


def _zero_rows(zbuf, rows, width):
    zv = jnp.zeros((16,), _f32)

    def _z(i, _):
        for j in range(width // 16):
            zbuf[i, pl.ds(j * 16, 16)] = zv
        return 0

    lax.fori_loop(0, rows, _z, 0)


def _init_acc(tables, zbuf, sid):
    for t in tables:
        for k in range(RPS // 208):
            pltpu.sync_copy(zbuf.at[pl.ds(0, 208)],
                            t.at[pl.ds(sid * RPS + k * 208, 208)])

    @pl.when(sid == NS - 1)
    def _():
        for t in tables:
            pltpu.sync_copy(zbuf.at[pl.ds(0, 16)], t.at[pl.ds(NS * RPS, 16)])


def _copy_out(acc, out, cid, sid):
    sl = pl.ds(sid * RPS, RPS)
    pltpu.sync_copy(acc.at[sl], out.at[cid, sl])

    @pl.when(sid == NS - 1)
    def _():
        tl = pl.ds(NS * RPS, 16)
        pltpu.sync_copy(acc.at[tl], out.at[cid, tl])


def _tcee_body(ea, we, be, ee_ref):
    ee_ref[...] = jnp.dot(ea[...], we[...],
                          preferred_element_type=_f32) + be[...]


_tc_ee = pl.pallas_call(
    _tcee_body,
    grid=(NBE,),
    in_specs=[
        pl.BlockSpec((BE, DE), lambda i: (i, 0)),
        pl.BlockSpec((DE, H), lambda i: (0, 0)),
        pl.BlockSpec((1, H), lambda i: (0, 0)),
    ],
    out_specs=pl.BlockSpec((BE, H), lambda i: (i, 0)),
    out_shape=jax.ShapeDtypeStruct((E, H), _f32),
)


def _edge_body(ee_hbm, row_hbm, col_hbm, oacc, ocnt,
               idxR, idxC, e0, e1, e2, e3, ones, zbuf, zbuf16,
               acc, cnt,
               l0, l1, l2, l3, r0, r1, r2, r3, c0, c1, c2, c3,
               n0, n1, n2, n3):
    cid = lax.axis_index("c")
    sid = lax.axis_index("s")
    w = cid * NS + sid
    base = w * EPW
    eb = (e0, e1, e2, e3)
    lsem = (l0, l1, l2, l3)
    rsem = (r0, r1, r2, r3)
    csem = (c0, c1, c2, c3)
    nsem = (n0, n1, n2, n3)

    _zero_rows(zbuf, 208, H)
    _zero_rows(zbuf16, 208, DE)
    _init_acc((acc,), zbuf, sid)
    _init_acc((cnt,), zbuf16, sid)

    ov = jnp.ones((16,), _f32)

    def _o(i, _):
        ones[i, :] = ov
        return 0

    lax.fori_loop(0, CK, _o, 0)

    pltpu.sync_copy(row_hbm.at[w], idxR)
    pltpu.sync_copy(col_hbm.at[w], idxC)
    plsc.subcore_barrier()

    def _load(c, b):
        n = TAIL if isinstance(c, int) and c >= FE else CK
        return pltpu.make_async_copy(
            ee_hbm.at[pl.ds(base + c * CK, n)],
            eb[b].at[pl.ds(0, n)], lsem[b])

    def _scatR(c, b, start):
        if start:
            pltpu.async_copy(eb[b], acc.at[idxR.at[c]], rsem[b], add=True)
        else:
            pltpu.make_async_copy(eb[b], acc.at[idxR.at[c]], rsem[b]).wait()

    def _scatC(c, b, start):
        if start:
            pltpu.async_copy(eb[b], acc.at[idxC.at[c]], csem[b], add=True)
        else:
            pltpu.make_async_copy(eb[b], acc.at[idxC.at[c]], csem[b]).wait()

    def _scatN(c, b, start):
        if start:
            pltpu.async_copy(ones, cnt.at[idxC.at[c]], nsem[b], add=True)
        else:
            pltpu.make_async_copy(ones, cnt.at[idxC.at[c]], nsem[b]).wait()

    for c in range(3):
        _load(c, c).start()

    def _step(c, j, start_next, wait_prev):
        b = j % 4
        b3 = (j + 3) % 4
        _load(c, b).wait()
        _scatR(c, b, True)
        _scatC(c, b, True)
        _scatN(c, b, True)
        if start_next:
            if wait_prev:
                _scatR(c - 1, b3, False)
                _scatC(c - 1, b3, False)
                _scatN(c - 1, b3, False)
            _load(c + 3, b3).start()

    _step(0, 0, True, False)
    for j in range(1, 4):
        _step(j, j, True, True)

    def _bodyq(q, _):
        cb = (q + 1) * 4
        for j in range(4):
            _step(cb + j, j, True, True)
        return 0

    lax.fori_loop(0, FE // 4 - 2, _bodyq, 0)

    for c in range(FE - 6, FE - 2):
        _step(c, c % 4, True, True)

    for c in range(FE - 2, FE + 1):
        _step(c, c % 4, False, False)

    for c in range(FE - 3, FE + 1):
        _scatR(c, c % 4, False)
        _scatC(c, c % 4, False)
        _scatN(c, c % 4, False)

    plsc.subcore_barrier()
    _copy_out(acc, oacc, cid, sid)
    _copy_out(cnt, ocnt, cid, sid)


_edge_pass = pl.kernel(
    _edge_body,
    out_type=[
        jax.ShapeDtypeStruct((NC, N, H), _f32),
        jax.ShapeDtypeStruct((NC, N, DE), _f32),
    ],
    mesh=_mesh,
    scratch_types=[
        pltpu.VMEM((NCH, CK), jnp.int32),
        pltpu.VMEM((NCH, CK), jnp.int32),
        pltpu.VMEM((CK, H), _f32),
        pltpu.VMEM((CK, H), _f32),
        pltpu.VMEM((CK, H), _f32),
        pltpu.VMEM((CK, H), _f32),
        pltpu.VMEM((CK, DE), _f32),
        pltpu.VMEM((208, H), _f32),
        pltpu.VMEM((208, DE), _f32),
        pltpu.VMEM_SHARED((ND, H), _f32),
        pltpu.VMEM_SHARED((ND, DE), _f32),
    ] + [pltpu.SemaphoreType.DMA] * 16,
    compiler_params=_sc_params,
)


def _conv_body(u_hbm, row_hbm, col_hbm, oacc,
               idxR, idxC, g0, g1, g2, g3, zbuf, acc,
               a0, a1, a2, a3, s0, s1, s2, s3):
    cid = lax.axis_index("c")
    sid = lax.axis_index("s")
    w = cid * NS + sid
    gb = (g0, g1, g2, g3)
    gsem = (a0, a1, a2, a3)
    ssem = (s0, s1, s2, s3)

    _zero_rows(zbuf, 208, H)
    _init_acc((acc,), zbuf, sid)

    pltpu.sync_copy(row_hbm.at[w], idxR)
    pltpu.sync_copy(col_hbm.at[w], idxC)
    plsc.subcore_barrier()

    def _gat(c, b):
        return pltpu.make_async_copy(u_hbm.at[idxR.at[c]], gb[b], gsem[b])

    def _scat(c, b, start):
        if start:
            pltpu.async_copy(gb[b], acc.at[idxC.at[c]], ssem[b], add=True)
        else:
            pltpu.make_async_copy(gb[b], acc.at[idxC.at[c]], ssem[b]).wait()

    for c in range(3):
        _gat(c, c).start()

    def _step(c, j, start_next, wait_prev):
        b = j % 4
        b3 = (j + 3) % 4
        _gat(c, b).wait()
        _scat(c, b, True)
        if start_next:
            if wait_prev:
                _scat(c - 1, b3, False)
            _gat(c + 3, b3).start()

    _step(0, 0, True, False)
    for j in range(1, 4):
        _step(j, j, True, True)

    def _bodyq(q, _):
        cb = (q + 1) * 4
        for j in range(4):
            c = cb + j
            _step(c, j, True, True)
        return 0

    lax.fori_loop(0, NCH // 4 - 2, _bodyq, 0)

    for j in range(4):
        c = NCH - 4 + j
        _step(c, j, j == 0, True)

    for j in range(4):
        _scat(NCH - 4 + j, j, False)

    plsc.subcore_barrier()
    _copy_out(acc, oacc, cid, sid)


_conv_pass = pl.kernel(
    _conv_body,
    out_type=jax.ShapeDtypeStruct((NC, N, H), _f32),
    mesh=_mesh,
    scratch_types=[
        pltpu.VMEM((NCH, CK), jnp.int32),
        pltpu.VMEM((NCH, CK), jnp.int32),
        pltpu.VMEM((CK, H), _f32),
        pltpu.VMEM((CK, H), _f32),
        pltpu.VMEM((CK, H), _f32),
        pltpu.VMEM((CK, H), _f32),
        pltpu.VMEM((208, H), _f32),
        pltpu.VMEM_SHARED((ND, H), _f32),
    ] + [pltpu.SemaphoreType.DMA] * 8,
    compiler_params=_sc_params,
)


def _tc1_body(x_ref, wn, bn, wc1, av, cnt, u1_ref, dinv_ref):
    agg = av[0] + av[1]
    deg = (cnt[0, :, 0:1] + cnt[1, :, 0:1]) + 1.0
    dinv = 1.0 / jnp.sqrt(deg)
    h = jnp.dot(x_ref[...], wn[...], preferred_element_type=_f32) + bn[...] + agg
    u1_ref[...] = dinv * jnp.dot(h, wc1[...], preferred_element_type=_f32)
    dinv_ref[...] = dinv


def _full(shape):
    return pl.BlockSpec(shape, lambda i: (0,) * len(shape))


def _rows(width):
    return pl.BlockSpec((BR, width), lambda i: (i, 0))


def _rows2(width):
    return pl.BlockSpec((NC, BR, width), lambda i: (0, i, 0))


_tc1 = pl.pallas_call(
    _tc1_body,
    grid=(NB,),
    in_specs=[
        _rows(DF), _full((DF, H)), _full((1, H)), _full((H, H)),
        _rows2(H), _rows2(DE),
    ],
    out_specs=[_rows(H), _rows(1)],
    out_shape=[
        jax.ShapeDtypeStruct((N, H), _f32),
        jax.ShapeDtypeStruct((N, 1), _f32),
    ],
)


def _tcmid_body(v, u, dinv, bc, wcn, x_ref, un_ref):
    xl = jnp.maximum(dinv[...] * ((v[0] + v[1]) + u[...]) + bc[...], 0.0)
    x_ref[...] = xl
    un_ref[...] = dinv[...] * jnp.dot(xl, wcn[...], preferred_element_type=_f32)


_tcmid = pl.pallas_call(
    _tcmid_body,
    grid=(NB,),
    in_specs=[_rows2(H), _rows(H), _rows(1), _full((1, H)), _full((H, H))],
    out_specs=[_rows(H), _rows(H)],
    out_shape=[
        jax.ShapeDtypeStruct((N, H), _f32),
        jax.ShapeDtypeStruct((N, H), _f32),
    ],
)


def _tcfin_body(v, u, dinv, bc3, x1, x2, bat,
                wk1a, wk1b, wk1c, bk1, wk2r, bk2,
                out_ref, s1, s2, s3, cnts):
    i = pl.program_id(0)
    x3 = jnp.maximum(dinv[...] * ((v[0] + v[1]) + u[...]) + bc3[...], 0.0)

    cols = lax.broadcasted_iota(jnp.int32, (BR, G), 1)
    oh = (bat[...] == cols).astype(_f32)

    @pl.when(i == 0)
    def _():
        s1[...] = jnp.zeros_like(s1)
        s2[...] = jnp.zeros_like(s2)
        s3[...] = jnp.zeros_like(s3)
        cnts[...] = jnp.zeros_like(cnts)

    dn = (((0,), (0,)), ((), ()))
    s1[...] += lax.dot_general(oh, x1[...], dn, preferred_element_type=_f32,
                               precision=_PH)
    s2[...] += lax.dot_general(oh, x2[...], dn, preferred_element_type=_f32,
                               precision=_PH)
    s3[...] += lax.dot_general(oh, x3, dn, preferred_element_type=_f32,
                               precision=_PH)
    cnts[...] += lax.dot_general(oh, jnp.ones((BR, 1), _f32), dn,
                                 preferred_element_type=_f32,
                                 precision=_PH)

    @pl.when(i == NB - 1)
    def _():
        c = jnp.maximum(cnts[...], 1.0)
        z = (jnp.dot(s1[...] / c, wk1a[...], preferred_element_type=_f32)
             + jnp.dot(s2[...] / c, wk1b[...], preferred_element_type=_f32)
             + jnp.dot(s3[...] / c, wk1c[...], preferred_element_type=_f32)
             + bk1[...])
        z = jnp.maximum(z, 0.0)
        o = jnp.sum(_bf16r(z) * _bf16r(wk2r[...]), axis=1)[None, :] + bk2[...]
        out_ref[...] = o


_tcfin = pl.pallas_call(
    _tcfin_body,
    grid=(NB,),
    in_specs=[
        _rows2(H), _rows(H), _rows(1), _full((1, H)),
        _rows(H), _rows(H), _rows(1),
        _full((H, H // 2)), _full((H, H // 2)), _full((H, H // 2)),
        _full((1, H // 2)), _full((1, H // 2)), _full((1, 1)),
    ],
    out_specs=_full((1, G)),
    out_shape=jax.ShapeDtypeStruct((1, G), _f32),
    scratch_shapes=[
        pltpu.VMEM((G, H), _f32),
        pltpu.VMEM((G, H), _f32),
        pltpu.VMEM((G, H), _f32),
        pltpu.VMEM((G, 1), _f32),
    ],
)


def kernel(x, edge_index, edge_attr, batch,
           Wn, bn, We, be, Wc1, bc1, Wc2, bc2, Wc3, bc3, Wk1, bk1, Wk2, bk2):
    row2 = edge_index[0].reshape(NW, EPW)
    col2 = edge_index[1].reshape(NW, EPW)
    rowg = jnp.pad(row2, ((0, 0), (0, PAD))).reshape(NW, NCH, CK)
    rowsc = jnp.pad(row2, ((0, 0), (0, PAD)), constant_values=N).reshape(NW, NCH, CK)
    colsc = jnp.pad(col2, ((0, 0), (0, PAD)), constant_values=N).reshape(NW, NCH, CK)

    ee = _tc_ee(edge_attr, We, be.reshape(1, H))
    av, cnt = _edge_pass(ee, rowsc, colsc)
    u1, dinv = _tc1(x, Wn, bn.reshape(1, H), Wc1, av, cnt)

    v = _conv_pass(u1, rowg, colsc)
    x1, u2 = _tcmid(v, u1, dinv, bc1.reshape(1, H), Wc2)
    v = _conv_pass(u2, rowg, colsc)
    x2, u3 = _tcmid(v, u2, dinv, bc2.reshape(1, H), Wc3)
    v = _conv_pass(u3, rowg, colsc)

    out = _tcfin(v, u3, dinv, bc3.reshape(1, H),
                 x1, x2, batch.reshape(N, 1),
                 Wk1[0:H], Wk1[H:2 * H], Wk1[2 * H:3 * H],
                 bk1.reshape(1, H // 2), Wk2.reshape(1, H // 2),
                 bk2.reshape(1, 1))
    return out.reshape(G)


# speedup vs baseline: 1.0065x; 1.0065x over previous
import jax
import jax.numpy as jnp
from jax import lax
from jax.experimental import pallas as pl
from jax.experimental.pallas import tpu as pltpu
from jax.experimental.pallas import tpu_sc as plsc

N = 10000
E = 320000
DF = 128
DE = 16
H = 64
G = 64

NC = 2
NS = 16
NW = NC * NS
EPW = E // NW
CK = 128
NCH = 80
PAD = NCH * CK - EPW
ND = N + 16
FE = EPW // CK
TAIL = EPW - FE * CK
RPS = 624
BR = 1000
NB = N // BR
BE = 4000
NBE = E // BE

_mesh = plsc.VectorSubcoreMesh(core_axis_name="c", subcore_axis_name="s")
_f32 = jnp.float32
_sc_params = pltpu.CompilerParams(use_tc_tiling_on_sc=False)
_PH = lax.Precision.HIGHEST


def _bf16r(v):
    return v.astype(jnp.bfloat16).astype(_f32)


def _zero_rows(zbuf, rows, width):
    zv = jnp.zeros((16,), _f32)

    def _z(i, _):
        for j in range(width // 16):
            zbuf[i, pl.ds(j * 16, 16)] = zv
        return 0

    lax.fori_loop(0, rows, _z, 0)


def _init_acc(tables, zbuf, sid):
    for t in tables:
        for k in range(RPS // 208):
            pltpu.sync_copy(zbuf.at[pl.ds(0, 208)],
                            t.at[pl.ds(sid * RPS + k * 208, 208)])

    @pl.when(sid == NS - 1)
    def _():
        for t in tables:
            pltpu.sync_copy(zbuf.at[pl.ds(0, 16)], t.at[pl.ds(NS * RPS, 16)])


def _copy_out(acc, out, cid, sid):
    sl = pl.ds(sid * RPS, RPS)
    pltpu.sync_copy(acc.at[sl], out.at[cid, sl])

    @pl.when(sid == NS - 1)
    def _():
        tl = pl.ds(NS * RPS, 16)
        pltpu.sync_copy(acc.at[tl], out.at[cid, tl])


def _tcee_body(ea, we, be, ee_ref):
    ee_ref[...] = jnp.dot(ea[...], we[...],
                          preferred_element_type=_f32) + be[...]


_tc_ee = pl.pallas_call(
    _tcee_body,
    grid=(NBE,),
    in_specs=[
        pl.BlockSpec((BE, DE), lambda i: (i, 0)),
        pl.BlockSpec((DE, H), lambda i: (0, 0)),
        pl.BlockSpec((1, H), lambda i: (0, 0)),
    ],
    out_specs=pl.BlockSpec((BE, H), lambda i: (i, 0)),
    out_shape=jax.ShapeDtypeStruct((E, H), _f32),
)


def _edge_body(ee_hbm, row_hbm, col_hbm, oacc, ocnt,
               idxR, idxC, e0, e1, e2, e3, ones, zbuf, zbuf16,
               acc, cnt,
               l0, l1, l2, l3, r0, r1, r2, r3, c0, c1, c2, c3,
               n0, n1, n2, n3):
    cid = lax.axis_index("c")
    sid = lax.axis_index("s")
    w = cid * NS + sid
    base = w * EPW
    eb = (e0, e1, e2, e3)
    lsem = (l0, l1, l2, l3)
    rsem = (r0, r1, r2, r3)
    csem = (c0, c1, c2, c3)
    nsem = (n0, n1, n2, n3)

    _zero_rows(zbuf, 208, H)
    _zero_rows(zbuf16, 208, DE)
    _init_acc((acc,), zbuf, sid)
    _init_acc((cnt,), zbuf16, sid)

    ov = jnp.ones((16,), _f32)

    def _o(i, _):
        ones[i, :] = ov
        return 0

    lax.fori_loop(0, CK, _o, 0)

    pltpu.sync_copy(row_hbm.at[w], idxR)
    pltpu.sync_copy(col_hbm.at[w], idxC)
    plsc.subcore_barrier()

    def _load(c, b):
        n = TAIL if isinstance(c, int) and c >= FE else CK
        return pltpu.make_async_copy(
            ee_hbm.at[pl.ds(base + c * CK, n)],
            eb[b].at[pl.ds(0, n)], lsem[b])

    def _scatR(c, b, start):
        if start:
            pltpu.async_copy(eb[b], acc.at[idxR.at[c]], rsem[b], add=True)
        else:
            pltpu.make_async_copy(eb[b], acc.at[idxR.at[c]], rsem[b]).wait()

    def _scatC(c, b, start):
        if start:
            pltpu.async_copy(eb[b], acc.at[idxC.at[c]], csem[b], add=True)
        else:
            pltpu.make_async_copy(eb[b], acc.at[idxC.at[c]], csem[b]).wait()

    def _scatN(c, b, start):
        if start:
            pltpu.async_copy(ones, cnt.at[idxC.at[c]], nsem[b], add=True)
        else:
            pltpu.make_async_copy(ones, cnt.at[idxC.at[c]], nsem[b]).wait()

    for c in range(3):
        _load(c, c).start()

    def _step(c, j, start_next, wait_prev):
        b = j % 4
        b3 = (j + 3) % 4
        _load(c, b).wait()
        _scatR(c, b, True)
        _scatC(c, b, True)
        _scatN(c, b, True)
        if start_next:
            if wait_prev:
                _scatR(c - 1, b3, False)
                _scatC(c - 1, b3, False)
                _scatN(c - 1, b3, False)
            _load(c + 3, b3).start()

    _step(0, 0, True, False)
    for j in range(1, 4):
        _step(j, j, True, True)

    def _bodyq(q, _):
        cb = (q + 1) * 4
        for j in range(4):
            _step(cb + j, j, True, True)
        return 0

    lax.fori_loop(0, FE // 4 - 2, _bodyq, 0)

    for c in range(FE - 6, FE - 2):
        _step(c, c % 4, True, True)

    for c in range(FE - 2, FE + 1):
        _step(c, c % 4, False, False)

    for c in range(FE - 3, FE + 1):
        _scatR(c, c % 4, False)
        _scatC(c, c % 4, False)
        _scatN(c, c % 4, False)

    plsc.subcore_barrier()
    _copy_out(acc, oacc, cid, sid)
    _copy_out(cnt, ocnt, cid, sid)


_edge_pass = pl.kernel(
    _edge_body,
    out_type=[
        jax.ShapeDtypeStruct((NC, N, H), _f32),
        jax.ShapeDtypeStruct((NC, N, DE), _f32),
    ],
    mesh=_mesh,
    scratch_types=[
        pltpu.VMEM((NCH, CK), jnp.int32),
        pltpu.VMEM((NCH, CK), jnp.int32),
        pltpu.VMEM((CK, H), _f32),
        pltpu.VMEM((CK, H), _f32),
        pltpu.VMEM((CK, H), _f32),
        pltpu.VMEM((CK, H), _f32),
        pltpu.VMEM((CK, DE), _f32),
        pltpu.VMEM((208, H), _f32),
        pltpu.VMEM((208, DE), _f32),
        pltpu.VMEM_SHARED((ND, H), _f32),
        pltpu.VMEM_SHARED((ND, DE), _f32),
    ] + [pltpu.SemaphoreType.DMA] * 16,
    compiler_params=_sc_params,
)


def _conv_body(u_hbm, row_hbm, col_hbm, oacc,
               idxR, idxC, g0, g1, g2, g3, g4, zbuf, acc,
               a0, a1, a2, a3, a4, s0, s1, s2, s3, s4):
    cid = lax.axis_index("c")
    sid = lax.axis_index("s")
    w = cid * NS + sid
    gb = (g0, g1, g2, g3, g4)
    gsem = (a0, a1, a2, a3, a4)
    ssem = (s0, s1, s2, s3, s4)

    _zero_rows(zbuf, 208, H)
    _init_acc((acc,), zbuf, sid)

    pltpu.sync_copy(row_hbm.at[w], idxR)
    pltpu.sync_copy(col_hbm.at[w], idxC)
    plsc.subcore_barrier()

    def _gat(c, b):
        return pltpu.make_async_copy(u_hbm.at[idxR.at[c]], gb[b], gsem[b])

    def _scat(c, b, start):
        if start:
            pltpu.async_copy(gb[b], acc.at[idxC.at[c]], ssem[b], add=True)
        else:
            pltpu.make_async_copy(gb[b], acc.at[idxC.at[c]], ssem[b]).wait()

    for c in range(4):
        _gat(c, c).start()

    def _step(c, j, start_next, wait_prev):
        b = j % 5
        b4 = (j + 4) % 5
        _gat(c, b).wait()
        _scat(c, b, True)
        if start_next:
            if wait_prev:
                _scat(c - 1, b4, False)
            _gat(c + 4, b4).start()

    _step(0, 0, True, False)
    for j in range(1, 5):
        _step(j, j, True, True)

    def _bodyq(q, _):
        cb = (q + 1) * 5
        for j in range(5):
            _step(cb + j, j, True, True)
        return 0

    lax.fori_loop(0, NCH // 5 - 2, _bodyq, 0)

    for j in range(5):
        _step(NCH - 5 + j, j, j == 0, True)

    for j in range(5):
        _scat(NCH - 5 + j, j, False)

    plsc.subcore_barrier()
    _copy_out(acc, oacc, cid, sid)


_conv_pass = pl.kernel(
    _conv_body,
    out_type=jax.ShapeDtypeStruct((NC, N, H), _f32),
    mesh=_mesh,
    scratch_types=[
        pltpu.VMEM((NCH, CK), jnp.int32),
        pltpu.VMEM((NCH, CK), jnp.int32),
    ] + [pltpu.VMEM((CK, H), _f32)] * 5 + [
        pltpu.VMEM((208, H), _f32),
        pltpu.VMEM_SHARED((ND, H), _f32),
    ] + [pltpu.SemaphoreType.DMA] * 10,
    compiler_params=_sc_params,
)


def _tc1_body(x_ref, wn, bn, wc1, av, cnt, u1_ref, dinv_ref):
    agg = av[0] + av[1]
    deg = (cnt[0, :, 0:1] + cnt[1, :, 0:1]) + 1.0
    dinv = 1.0 / jnp.sqrt(deg)
    h = jnp.dot(x_ref[...], wn[...], preferred_element_type=_f32) + bn[...] + agg
    u1_ref[...] = dinv * jnp.dot(h, wc1[...], preferred_element_type=_f32)
    dinv_ref[...] = dinv


def _full(shape):
    return pl.BlockSpec(shape, lambda i: (0,) * len(shape))


def _rows(width):
    return pl.BlockSpec((BR, width), lambda i: (i, 0))


def _rows2(width):
    return pl.BlockSpec((NC, BR, width), lambda i: (0, i, 0))


_tc1 = pl.pallas_call(
    _tc1_body,
    grid=(NB,),
    in_specs=[
        _rows(DF), _full((DF, H)), _full((1, H)), _full((H, H)),
        _rows2(H), _rows2(DE),
    ],
    out_specs=[_rows(H), _rows(1)],
    out_shape=[
        jax.ShapeDtypeStruct((N, H), _f32),
        jax.ShapeDtypeStruct((N, 1), _f32),
    ],
)


def _tcmid_body(v, u, dinv, bc, wcn, x_ref, un_ref):
    xl = jnp.maximum(dinv[...] * ((v[0] + v[1]) + u[...]) + bc[...], 0.0)
    x_ref[...] = xl
    un_ref[...] = dinv[...] * jnp.dot(xl, wcn[...], preferred_element_type=_f32)


_tcmid = pl.pallas_call(
    _tcmid_body,
    grid=(NB,),
    in_specs=[_rows2(H), _rows(H), _rows(1), _full((1, H)), _full((H, H))],
    out_specs=[_rows(H), _rows(H)],
    out_shape=[
        jax.ShapeDtypeStruct((N, H), _f32),
        jax.ShapeDtypeStruct((N, H), _f32),
    ],
)


def _tcfin_body(v, u, dinv, bc3, x1, x2, bat,
                wk1a, wk1b, wk1c, bk1, wk2r, bk2,
                out_ref, s1, s2, s3, cnts):
    i = pl.program_id(0)
    x3 = jnp.maximum(dinv[...] * ((v[0] + v[1]) + u[...]) + bc3[...], 0.0)

    cols = lax.broadcasted_iota(jnp.int32, (BR, G), 1)
    oh = (bat[...] == cols).astype(_f32)

    @pl.when(i == 0)
    def _():
        s1[...] = jnp.zeros_like(s1)
        s2[...] = jnp.zeros_like(s2)
        s3[...] = jnp.zeros_like(s3)
        cnts[...] = jnp.zeros_like(cnts)

    dn = (((0,), (0,)), ((), ()))
    s1[...] += lax.dot_general(oh, x1[...], dn, preferred_element_type=_f32,
                               precision=_PH)
    s2[...] += lax.dot_general(oh, x2[...], dn, preferred_element_type=_f32,
                               precision=_PH)
    s3[...] += lax.dot_general(oh, x3, dn, preferred_element_type=_f32,
                               precision=_PH)
    cnts[...] += lax.dot_general(oh, jnp.ones((BR, 1), _f32), dn,
                                 preferred_element_type=_f32,
                                 precision=_PH)

    @pl.when(i == NB - 1)
    def _():
        c = jnp.maximum(cnts[...], 1.0)
        z = (jnp.dot(s1[...] / c, wk1a[...], preferred_element_type=_f32)
             + jnp.dot(s2[...] / c, wk1b[...], preferred_element_type=_f32)
             + jnp.dot(s3[...] / c, wk1c[...], preferred_element_type=_f32)
             + bk1[...])
        z = jnp.maximum(z, 0.0)
        o = jnp.sum(_bf16r(z) * _bf16r(wk2r[...]), axis=1)[None, :] + bk2[...]
        out_ref[...] = o


_tcfin = pl.pallas_call(
    _tcfin_body,
    grid=(NB,),
    in_specs=[
        _rows2(H), _rows(H), _rows(1), _full((1, H)),
        _rows(H), _rows(H), _rows(1),
        _full((H, H // 2)), _full((H, H // 2)), _full((H, H // 2)),
        _full((1, H // 2)), _full((1, H // 2)), _full((1, 1)),
    ],
    out_specs=_full((1, G)),
    out_shape=jax.ShapeDtypeStruct((1, G), _f32),
    scratch_shapes=[
        pltpu.VMEM((G, H), _f32),
        pltpu.VMEM((G, H), _f32),
        pltpu.VMEM((G, H), _f32),
        pltpu.VMEM((G, 1), _f32),
    ],
)


def kernel(x, edge_index, edge_attr, batch,
           Wn, bn, We, be, Wc1, bc1, Wc2, bc2, Wc3, bc3, Wk1, bk1, Wk2, bk2):
    row2 = edge_index[0].reshape(NW, EPW)
    col2 = edge_index[1].reshape(NW, EPW)
    rowg = jnp.pad(row2, ((0, 0), (0, PAD))).reshape(NW, NCH, CK)
    rowsc = jnp.pad(row2, ((0, 0), (0, PAD)), constant_values=N).reshape(NW, NCH, CK)
    colsc = jnp.pad(col2, ((0, 0), (0, PAD)), constant_values=N).reshape(NW, NCH, CK)

    ee = _tc_ee(edge_attr, We, be.reshape(1, H))
    av, cnt = _edge_pass(ee, rowsc, colsc)
    u1, dinv = _tc1(x, Wn, bn.reshape(1, H), Wc1, av, cnt)

    v = _conv_pass(u1, rowg, colsc)
    x1, u2 = _tcmid(v, u1, dinv, bc1.reshape(1, H), Wc2)
    v = _conv_pass(u2, rowg, colsc)
    x2, u3 = _tcmid(v, u2, dinv, bc2.reshape(1, H), Wc3)
    v = _conv_pass(u3, rowg, colsc)

    out = _tcfin(v, u3, dinv, bc3.reshape(1, H),
                 x1, x2, batch.reshape(N, 1),
                 Wk1[0:H], Wk1[H:2 * H], Wk1[2 * H:3 * H],
                 bk1.reshape(1, H // 2), Wk2.reshape(1, H // 2),
                 bk2.reshape(1, 1))
    return out.reshape(G)


# speedup vs baseline: 1.0066x; 1.0001x over previous
import jax
import jax.numpy as jnp
from jax import lax
from jax.experimental import pallas as pl
from jax.experimental.pallas import tpu as pltpu
from jax.experimental.pallas import tpu_sc as plsc

N = 10000
E = 320000
DF = 128
DE = 16
H = 64
G = 64

NC = 2
NS = 16
NW = NC * NS
EPW = E // NW
CK = 128
NCH = 80
PAD = NCH * CK - EPW
ND = N + 16
FE = EPW // CK
TAIL = EPW - FE * CK
RPS = 624
BR = 1000
NB = N // BR
BE = 4000
NBE = E // BE

_mesh = plsc.VectorSubcoreMesh(core_axis_name="c", subcore_axis_name="s")
_f32 = jnp.float32
_sc_params = pltpu.CompilerParams(use_tc_tiling_on_sc=False)
_PH = lax.Precision.HIGHEST


def _zero_rows(zbuf, rows, width):
    zv = jnp.zeros((16,), _f32)

    def _z(i, _):
        for j in range(width // 16):
            zbuf[i, pl.ds(j * 16, 16)] = zv
        return 0

    lax.fori_loop(0, rows, _z, 0)


def _init_acc(tables, zbuf, sid):
    for t in tables:
        for k in range(RPS // 208):
            pltpu.sync_copy(zbuf.at[pl.ds(0, 208)],
                            t.at[pl.ds(sid * RPS + k * 208, 208)])

    @pl.when(sid == NS - 1)
    def _():
        for t in tables:
            pltpu.sync_copy(zbuf.at[pl.ds(0, 16)], t.at[pl.ds(NS * RPS, 16)])


def _copy_out(acc, out, cid, sid):
    sl = pl.ds(sid * RPS, RPS)
    pltpu.sync_copy(acc.at[sl], out.at[cid, sl])

    @pl.when(sid == NS - 1)
    def _():
        tl = pl.ds(NS * RPS, 16)
        pltpu.sync_copy(acc.at[tl], out.at[cid, tl])


def _tcee_body(ea, we, be, ee_ref):
    ee_ref[...] = jnp.dot(ea[...], we[...],
                          preferred_element_type=_f32) + be[...]


_tc_ee = pl.pallas_call(
    _tcee_body,
    grid=(NBE,),
    in_specs=[
        pl.BlockSpec((BE, DE), lambda i: (i, 0)),
        pl.BlockSpec((DE, H), lambda i: (0, 0)),
        pl.BlockSpec((1, H), lambda i: (0, 0)),
    ],
    out_specs=pl.BlockSpec((BE, H), lambda i: (i, 0)),
    out_shape=jax.ShapeDtypeStruct((E, H), _f32),
)


def _edge_body(ee_hbm, row_hbm, col_hbm, oacc, ocnt,
               idxR, idxC, e0, e1, e2, e3, ones, zbuf, zbuf16,
               acc, cnt,
               l0, l1, l2, l3, r0, r1, r2, r3, c0, c1, c2, c3,
               n0, n1, n2, n3):
    cid = lax.axis_index("c")
    sid = lax.axis_index("s")
    w = cid * NS + sid
    base = w * EPW
    eb = (e0, e1, e2, e3)
    lsem = (l0, l1, l2, l3)
    rsem = (r0, r1, r2, r3)
    csem = (c0, c1, c2, c3)
    nsem = (n0, n1, n2, n3)

    _zero_rows(zbuf, 208, H)
    _zero_rows(zbuf16, 208, DE)
    _init_acc((acc,), zbuf, sid)
    _init_acc((cnt,), zbuf16, sid)

    ov = jnp.ones((16,), _f32)

    def _o(i, _):
        ones[i, :] = ov
        return 0

    lax.fori_loop(0, CK, _o, 0)

    pltpu.sync_copy(row_hbm.at[w], idxR)
    pltpu.sync_copy(col_hbm.at[w], idxC)
    plsc.subcore_barrier()

    def _load(c, b):
        n = TAIL if isinstance(c, int) and c >= FE else CK
        return pltpu.make_async_copy(
            ee_hbm.at[pl.ds(base + c * CK, n)],
            eb[b].at[pl.ds(0, n)], lsem[b])

    def _scatR(c, b, start):
        if start:
            pltpu.async_copy(eb[b], acc.at[idxR.at[c]], rsem[b], add=True)
        else:
            pltpu.make_async_copy(eb[b], acc.at[idxR.at[c]], rsem[b]).wait()

    def _scatC(c, b, start):
        if start:
            pltpu.async_copy(eb[b], acc.at[idxC.at[c]], csem[b], add=True)
        else:
            pltpu.make_async_copy(eb[b], acc.at[idxC.at[c]], csem[b]).wait()

    def _scatN(c, b, start):
        if start:
            pltpu.async_copy(ones, cnt.at[idxC.at[c]], nsem[b], add=True)
        else:
            pltpu.make_async_copy(ones, cnt.at[idxC.at[c]], nsem[b]).wait()

    for c in range(3):
        _load(c, c).start()

    def _step(c, j, start_next, wait_prev):
        b = j % 4
        b3 = (j + 3) % 4
        _load(c, b).wait()
        _scatR(c, b, True)
        _scatC(c, b, True)
        _scatN(c, b, True)
        if start_next:
            if wait_prev:
                _scatR(c - 1, b3, False)
                _scatC(c - 1, b3, False)
                _scatN(c - 1, b3, False)
            _load(c + 3, b3).start()

    _step(0, 0, True, False)
    for j in range(1, 4):
        _step(j, j, True, True)

    def _bodyq(q, _):
        cb = (q + 1) * 4
        for j in range(4):
            _step(cb + j, j, True, True)
        return 0

    lax.fori_loop(0, FE // 4 - 2, _bodyq, 0)

    for c in range(FE - 6, FE - 2):
        _step(c, c % 4, True, True)

    for c in range(FE - 2, FE + 1):
        _step(c, c % 4, False, False)

    for c in range(FE - 3, FE + 1):
        _scatR(c, c % 4, False)
        _scatC(c, c % 4, False)
        _scatN(c, c % 4, False)

    plsc.subcore_barrier()
    _copy_out(acc, oacc, cid, sid)
    _copy_out(cnt, ocnt, cid, sid)


_edge_pass = pl.kernel(
    _edge_body,
    out_type=[
        jax.ShapeDtypeStruct((NC, N, H), _f32),
        jax.ShapeDtypeStruct((NC, N, DE), _f32),
    ],
    mesh=_mesh,
    scratch_types=[
        pltpu.VMEM((NCH, CK), jnp.int32),
        pltpu.VMEM((NCH, CK), jnp.int32),
        pltpu.VMEM((CK, H), _f32),
        pltpu.VMEM((CK, H), _f32),
        pltpu.VMEM((CK, H), _f32),
        pltpu.VMEM((CK, H), _f32),
        pltpu.VMEM((CK, DE), _f32),
        pltpu.VMEM((208, H), _f32),
        pltpu.VMEM((208, DE), _f32),
        pltpu.VMEM_SHARED((ND, H), _f32),
        pltpu.VMEM_SHARED((ND, DE), _f32),
    ] + [pltpu.SemaphoreType.DMA] * 16,
    compiler_params=_sc_params,
)


def _conv_body(u_hbm, row_hbm, col_hbm, oacc,
               idxR, idxC, g0, g1, g2, g3, g4, zbuf, acc,
               a0, a1, a2, a3, a4, s0, s1, s2, s3, s4):
    cid = lax.axis_index("c")
    sid = lax.axis_index("s")
    w = cid * NS + sid
    gb = (g0, g1, g2, g3, g4)
    gsem = (a0, a1, a2, a3, a4)
    ssem = (s0, s1, s2, s3, s4)

    _zero_rows(zbuf, 208, H)
    _init_acc((acc,), zbuf, sid)

    pltpu.sync_copy(row_hbm.at[w], idxR)
    pltpu.sync_copy(col_hbm.at[w], idxC)
    plsc.subcore_barrier()

    def _gat(c, b):
        return pltpu.make_async_copy(u_hbm.at[idxR.at[c]], gb[b], gsem[b])

    def _scat(c, b, start):
        if start:
            pltpu.async_copy(gb[b], acc.at[idxC.at[c]], ssem[b], add=True)
        else:
            pltpu.make_async_copy(gb[b], acc.at[idxC.at[c]], ssem[b]).wait()

    for c in range(4):
        _gat(c, c).start()

    def _step(c, j, start_next, wait_prev):
        b = j % 5
        b4 = (j + 4) % 5
        _gat(c, b).wait()
        _scat(c, b, True)
        if start_next:
            if wait_prev:
                _scat(c - 1, b4, False)
            _gat(c + 4, b4).start()

    _step(0, 0, True, False)
    for j in range(1, 5):
        _step(j, j, True, True)

    def _bodyq(q, _):
        cb = (q + 1) * 5
        for j in range(5):
            _step(cb + j, j, True, True)
        return 0

    lax.fori_loop(0, NCH // 5 - 2, _bodyq, 0)

    for j in range(5):
        _step(NCH - 5 + j, j, j == 0, True)

    for j in range(5):
        _scat(NCH - 5 + j, j, False)

    plsc.subcore_barrier()
    _copy_out(acc, oacc, cid, sid)


_conv_pass = pl.kernel(
    _conv_body,
    out_type=jax.ShapeDtypeStruct((NC, N, H), _f32),
    mesh=_mesh,
    scratch_types=[
        pltpu.VMEM((NCH, CK), jnp.int32),
        pltpu.VMEM((NCH, CK), jnp.int32),
    ] + [pltpu.VMEM((CK, H), _f32)] * 5 + [
        pltpu.VMEM((208, H), _f32),
        pltpu.VMEM_SHARED((ND, H), _f32),
    ] + [pltpu.SemaphoreType.DMA] * 10,
    compiler_params=_sc_params,
)


def _tc1_body(x_ref, wn, bn, wc1, av, cnt, u1_ref, dinv_ref):
    agg = av[0] + av[1]
    deg = (cnt[0, :, 0:1] + cnt[1, :, 0:1]) + 1.0
    dinv = 1.0 / jnp.sqrt(deg)
    h = jnp.dot(x_ref[...], wn[...], preferred_element_type=_f32) + bn[...] + agg
    u1_ref[...] = dinv * jnp.dot(h, wc1[...], preferred_element_type=_f32)
    dinv_ref[...] = dinv


def _full(shape):
    return pl.BlockSpec(shape, lambda i: (0,) * len(shape))


def _rows(width):
    return pl.BlockSpec((BR, width), lambda i: (i, 0))


def _rows2(width):
    return pl.BlockSpec((NC, BR, width), lambda i: (0, i, 0))


_tc1 = pl.pallas_call(
    _tc1_body,
    grid=(NB,),
    in_specs=[
        _rows(DF), _full((DF, H)), _full((1, H)), _full((H, H)),
        _rows2(H), _rows2(DE),
    ],
    out_specs=[_rows(H), _rows(1)],
    out_shape=[
        jax.ShapeDtypeStruct((N, H), _f32),
        jax.ShapeDtypeStruct((N, 1), _f32),
    ],
)


def _tcmid_body(v, u, dinv, bc, wcn, x_ref, un_ref):
    xl = jnp.maximum(dinv[...] * ((v[0] + v[1]) + u[...]) + bc[...], 0.0)
    x_ref[...] = xl
    un_ref[...] = dinv[...] * jnp.dot(xl, wcn[...], preferred_element_type=_f32)


_tcmid = pl.pallas_call(
    _tcmid_body,
    grid=(NB,),
    in_specs=[_rows2(H), _rows(H), _rows(1), _full((1, H)), _full((H, H))],
    out_specs=[_rows(H), _rows(H)],
    out_shape=[
        jax.ShapeDtypeStruct((N, H), _f32),
        jax.ShapeDtypeStruct((N, H), _f32),
    ],
)


def _tcfin_body(v, u, dinv, bc3, x1, x2, bat,
                wk1a, wk1b, wk1c, bk1, wk2c, bk2,
                out_ref, s1, s2, s3, cnts):
    i = pl.program_id(0)
    x3 = jnp.maximum(dinv[...] * ((v[0] + v[1]) + u[...]) + bc3[...], 0.0)

    cols = lax.broadcasted_iota(jnp.int32, (BR, G), 1)
    oh = (bat[...] == cols).astype(_f32)

    @pl.when(i == 0)
    def _():
        s1[...] = jnp.zeros_like(s1)
        s2[...] = jnp.zeros_like(s2)
        s3[...] = jnp.zeros_like(s3)
        cnts[...] = jnp.zeros_like(cnts)

    dn = (((0,), (0,)), ((), ()))
    s1[...] += lax.dot_general(oh, x1[...], dn, preferred_element_type=_f32,
                               precision=_PH)
    s2[...] += lax.dot_general(oh, x2[...], dn, preferred_element_type=_f32,
                               precision=_PH)
    s3[...] += lax.dot_general(oh, x3, dn, preferred_element_type=_f32,
                               precision=_PH)
    cnts[...] += lax.dot_general(oh, jnp.ones((BR, 1), _f32), dn,
                                 preferred_element_type=_f32,
                                 precision=_PH)

    @pl.when(i == NB - 1)
    def _():
        c = jnp.maximum(cnts[...], 1.0)
        z = (jnp.dot(s1[...] / c, wk1a[...], preferred_element_type=_f32)
             + jnp.dot(s2[...] / c, wk1b[...], preferred_element_type=_f32)
             + jnp.dot(s3[...] / c, wk1c[...], preferred_element_type=_f32)
             + bk1[...])
        z = jnp.maximum(z, 0.0)
        out_ref[...] = jnp.dot(z, wk2c[...],
                               preferred_element_type=_f32) + bk2[...]


_tcfin = pl.pallas_call(
    _tcfin_body,
    grid=(NB,),
    in_specs=[
        _rows2(H), _rows(H), _rows(1), _full((1, H)),
        _rows(H), _rows(H), _rows(1),
        _full((H, H // 2)), _full((H, H // 2)), _full((H, H // 2)),
        _full((1, H // 2)), _full((H // 2, 1)), _full((1, 1)),
    ],
    out_specs=_full((G, 1)),
    out_shape=jax.ShapeDtypeStruct((G, 1), _f32),
    scratch_shapes=[
        pltpu.VMEM((G, H), _f32),
        pltpu.VMEM((G, H), _f32),
        pltpu.VMEM((G, H), _f32),
        pltpu.VMEM((G, 1), _f32),
    ],
)


def kernel(x, edge_index, edge_attr, batch,
           Wn, bn, We, be, Wc1, bc1, Wc2, bc2, Wc3, bc3, Wk1, bk1, Wk2, bk2):
    row2 = edge_index[0].reshape(NW, EPW)
    col2 = edge_index[1].reshape(NW, EPW)
    rowg = jnp.pad(row2, ((0, 0), (0, PAD))).reshape(NW, NCH, CK)
    rowsc = jnp.pad(row2, ((0, 0), (0, PAD)), constant_values=N).reshape(NW, NCH, CK)
    colsc = jnp.pad(col2, ((0, 0), (0, PAD)), constant_values=N).reshape(NW, NCH, CK)

    ee = _tc_ee(edge_attr, We, be.reshape(1, H))
    av, cnt = _edge_pass(ee, rowsc, colsc)
    u1, dinv = _tc1(x, Wn, bn.reshape(1, H), Wc1, av, cnt)

    v = _conv_pass(u1, rowg, colsc)
    x1, u2 = _tcmid(v, u1, dinv, bc1.reshape(1, H), Wc2)
    v = _conv_pass(u2, rowg, colsc)
    x2, u3 = _tcmid(v, u2, dinv, bc2.reshape(1, H), Wc3)
    v = _conv_pass(u3, rowg, colsc)

    out = _tcfin(v, u3, dinv, bc3.reshape(1, H),
                 x1, x2, batch.reshape(N, 1),
                 Wk1[0:H], Wk1[H:2 * H], Wk1[2 * H:3 * H],
                 bk1.reshape(1, H // 2), Wk2,
                 bk2.reshape(1, 1))
    return out.reshape(G)
